# baseline (speedup 1.0000x reference)
import jax
import jax.numpy as jnp
from jax.experimental import pallas as pl
from jax.experimental.pallas import tpu as pltpu

_BLOCK = 1000


def _leaky(x):
    return jnp.where(x > 0, x, 0.01 * x)


def _fused_mlp(small_ref, tweet_ref, des_ref,
               w_small_ref, w_tweet_ref, w_des_ref, b_cat_ref,
               w_lin1_ref, b_lin1_ref, w_o1_ref, b_o1_ref,
               w_o2_ref, b_o2_ref, out_ref):
    pre = jnp.dot(tweet_ref[:], w_tweet_ref[:], preferred_element_type=jnp.float32)
    pre = pre + jnp.dot(des_ref[:], w_des_ref[:], preferred_element_type=jnp.float32)
    pre = pre + jnp.dot(small_ref[:], w_small_ref[:], preferred_element_type=jnp.float32)
    user = _leaky(pre + b_cat_ref[:])
    user = _leaky(jnp.dot(user, w_lin1_ref[:], preferred_element_type=jnp.float32)
                  + b_lin1_ref[:])
    u2 = _leaky(jnp.dot(user, w_o1_ref[:], preferred_element_type=jnp.float32)
                + b_o1_ref[:])
    out_ref[:] = (jnp.dot(u2, w_o2_ref[:], preferred_element_type=jnp.float32)
                  + b_o2_ref[:])


def kernel(des_features, tweet_features, prop_features, cat_features,
           edge_index, edge_type,
           W_num, b_num, W_bool, b_bool, W_tweet, b_tweet, W_des, b_des,
           W_lin1, b_lin1, W_gcn, b_gcn, W_out1, b_out1, W_out2, b_out2):
    n = des_features.shape[0]
    d_txt = des_features.shape[1]
    h = W_num.shape[1]
    lc = W_lin1.shape[0]
    oc1 = W_out1.shape[1]
    oc2 = W_out2.shape[1]
    f32 = jnp.float32

    small = jnp.concatenate(
        [prop_features, cat_features,
         jnp.zeros((n, 2), f32)], axis=1)

    w_small = jnp.zeros((8, lc), f32)
    w_small = w_small.at[0:5, 0:h].set(W_num)
    w_small = w_small.at[5:6, h:2 * h].set(W_bool)
    w_tweet_p = jnp.zeros((d_txt, lc), f32).at[:, 2 * h:3 * h].set(W_tweet)
    w_des_p = jnp.zeros((d_txt, lc), f32).at[:, 3 * h:4 * h].set(W_des)
    b_cat = jnp.concatenate([b_num, b_bool, b_tweet, b_des]).reshape(1, lc)

    grid = (n // _BLOCK,)
    row_blk = lambda i: (i, 0)
    whole = lambda i: (0, 0)

    out = pl.pallas_call(
        _fused_mlp,
        grid=grid,
        in_specs=[
            pl.BlockSpec((_BLOCK, 8), row_blk),
            pl.BlockSpec((_BLOCK, d_txt), row_blk),
            pl.BlockSpec((_BLOCK, d_txt), row_blk),
            pl.BlockSpec((8, lc), whole),
            pl.BlockSpec((d_txt, lc), whole),
            pl.BlockSpec((d_txt, lc), whole),
            pl.BlockSpec((1, lc), whole),
            pl.BlockSpec((lc, lc), whole),
            pl.BlockSpec((1, lc), whole),
            pl.BlockSpec((lc, oc1), whole),
            pl.BlockSpec((1, oc1), whole),
            pl.BlockSpec((oc1, oc2), whole),
            pl.BlockSpec((1, oc2), whole),
        ],
        out_specs=pl.BlockSpec((_BLOCK, oc2), row_blk),
        out_shape=jax.ShapeDtypeStruct((n, oc2), f32),
        compiler_params=pltpu.CompilerParams(
            dimension_semantics=("arbitrary",),
        ),
    )(small, tweet_features, des_features,
      w_small, w_tweet_p, w_des_p, b_cat,
      W_lin1, b_lin1.reshape(1, lc),
      W_out1, b_out1.reshape(1, oc1),
      W_out2, b_out2.reshape(1, oc2))
    return out
